# baseline (speedup 1.0000x reference)
import functools
import math

import jax
import jax.numpy as jnp
from jax import lax
from jax.experimental import pallas as pl
from jax.experimental.pallas import tpu as pltpu
from jax.experimental.pallas import tpu_sc as plsc

N = 10000
E = 320000
H = 128

INV_SQRT_3 = 1.0 / math.sqrt(3.0)
INV_SQRT_H = 1.0 / math.sqrt(float(H))
INV_SQRT_2 = 1.0 / math.sqrt(2.0)

F32 = jnp.float32

NC = 2
NS = 16
NW = NC * NS
NCHUNK = 1
EC = E // NCHUNK
GW = 40
SW = 80
EPW = EC // NW
GWIN = EPW // GW
EPT = EC // NS
SWIN = EPT // SW
RPT = 1000
NWRITERS = N // RPT

_sc_mesh = plsc.VectorSubcoreMesh(core_axis_name="c", subcore_axis_name="s")


def _node_mlp_body(x_ref, vec_ref, g_ref, bb_ref, w1_ref, b1_ref, w2_ref,
                   b2_ref, out_ref):
    x = x_ref[...]
    mu = jnp.mean(x, axis=-1, keepdims=True)
    var = jnp.mean((x - mu) ** 2, axis=-1, keepdims=True)
    xh = (x - mu) * lax.rsqrt(var + 1e-5) * g_ref[...] + bb_ref[...]
    h = jnp.dot(xh, w1_ref[...], preferred_element_type=F32) + b1_ref[...]
    h = jax.nn.silu(h) * (1.0 / 0.6)
    out_ref[:, :3 * H] = jnp.dot(h, w2_ref[...], preferred_element_type=F32) + b2_ref[...]
    out_ref[:, 3 * H:] = vec_ref[...]


def _node_mlp(x, vec2d, ln_g, ln_b, W1, b1, W2, b2):
    TN = 2000
    grid = (N // TN,)
    return pl.pallas_call(
        _node_mlp_body,
        grid=grid,
        in_specs=[
            pl.BlockSpec((TN, H), lambda i: (i, 0)),
            pl.BlockSpec((TN, 3 * H), lambda i: (i, 0)),
            pl.BlockSpec((1, H), lambda i: (0, 0)),
            pl.BlockSpec((1, H), lambda i: (0, 0)),
            pl.BlockSpec((H, H), lambda i: (0, 0)),
            pl.BlockSpec((1, H), lambda i: (0, 0)),
            pl.BlockSpec((H, 3 * H), lambda i: (0, 0)),
            pl.BlockSpec((1, 3 * H), lambda i: (0, 0)),
        ],
        out_specs=pl.BlockSpec((TN, 6 * H), lambda i: (i, 0)),
        out_shape=jax.ShapeDtypeStruct((N, 6 * H), F32),
    )(x, vec2d, ln_g.reshape(1, H), ln_b.reshape(1, H), W1, b1.reshape(1, H),
      W2, b2.reshape(1, 3 * H))


@functools.partial(
    pl.kernel,
    mesh=_sc_mesh,
    out_type=jax.ShapeDtypeStruct((EC, 6 * H), F32),
    scratch_types=[
        pltpu.VMEM((GWIN, GW), jnp.int32),
        pltpu.VMEM((2, GW, 6 * H), F32),
        pltpu.SemaphoreType.DMA,
        pltpu.SemaphoreType.DMA,
        pltpu.SemaphoreType.DMA,
        pltpu.SemaphoreType.DMA,
    ],
)
def _sc_gather(table, src_hbm, out, idx_v, rows_v, gsem0, gsem1, ssem0,
               ssem1):
    wid = lax.axis_index("s") * NC + lax.axis_index("c")
    pltpu.sync_copy(src_hbm.at[wid], idx_v)
    base = wid * EPW
    gsems = (gsem0, gsem1)
    ssems = (ssem0, ssem1)

    def start_gather(w, buf):
        pltpu.async_copy(table.at[idx_v.at[w]], rows_v.at[buf], gsems[buf])

    def drain_gather(buf):
        pltpu.make_async_copy(table.at[idx_v.at[0]], rows_v.at[buf],
                              gsems[buf]).wait()

    def drain_store(buf):
        pltpu.make_async_copy(rows_v.at[buf], out.at[pl.ds(base, GW)],
                              ssems[buf]).wait()

    def win(w, buf, guard):
        if guard is not False:
            def adv():
                drain_store(1 - buf)
                start_gather(w + 1, 1 - buf)

            if guard is True:
                adv()
            else:
                pl.when(guard)(adv)
        drain_gather(buf)
        pltpu.async_copy(rows_v.at[buf], out.at[pl.ds(base + w * GW, GW)],
                         ssems[buf])

    def pair(g, carry):
        win(2 * g, 0, g >= 1)
        win(2 * g + 1, 1, g < GWIN // 2 - 1)
        return carry

    start_gather(0, 0)
    start_gather(1, 1)
    lax.fori_loop(0, GWIN // 2, pair, 0)
    drain_store(1)
    drain_store(0)


def _edge_body(ee_ref, xv_ref, ev_ref, wr_ref, br_ref, pay_ref):
    rbfh = jnp.dot(ee_ref[...], wr_ref[...], preferred_element_type=F32) + br_ref[...]
    xv = xv_ref[...]
    m = xv[:, :3 * H] * rbfh
    mx = m[:, :H]
    m2 = m[:, H:2 * H] * INV_SQRT_3
    m3 = m[:, 2 * H:]
    ev = ev_ref[...]
    pay_ref[0] = mx
    for c in range(3):
        pay_ref[c + 1] = (xv[:, (3 + c) * H:(4 + c) * H] * m2
                          + m3 * ev[:, c:c + 1]) * INV_SQRT_H


def _edge_compute(edge_embed, xhvec_src, edge_vec, Wr, br):
    TE = 1280
    grid = (EC // TE,)
    return pl.pallas_call(
        _edge_body,
        grid=grid,
        in_specs=[
            pl.BlockSpec((TE, H), lambda i: (i, 0)),
            pl.BlockSpec((TE, 6 * H), lambda i: (i, 0)),
            pl.BlockSpec((TE, 3), lambda i: (i, 0)),
            pl.BlockSpec((H, 3 * H), lambda i: (0, 0)),
            pl.BlockSpec((1, 3 * H), lambda i: (0, 0)),
        ],
        out_specs=pl.BlockSpec((4, TE, H), lambda i: (0, i, 0)),
        out_shape=jax.ShapeDtypeStruct((4, EC, H), F32),
    )(edge_embed, xhvec_src, edge_vec, Wr, br.reshape(1, 3 * H))


@functools.partial(
    pl.kernel,
    mesh=_sc_mesh,
    out_type=jax.ShapeDtypeStruct((4, N, H), F32),
    scratch_types=[
        pltpu.VMEM((2, SW), jnp.int32),
        pltpu.VMEM((2, SW, H), F32),
        pltpu.VMEM_SHARED((N, H), F32),
        pltpu.SemaphoreType.DMA,
        pltpu.SemaphoreType.DMA,
    ],
)
def _sc_scatter(pay_hbm, dst_hbm, zeros_hbm, out_hbm, idx_v, upd_v, acc_sh,
                lsem0, lsem1):
    cid = lax.axis_index("c")
    sid = lax.axis_index("s")
    row0 = sid * RPT
    lsems = (lsem0, lsem1)

    def process(slab):
        ebase = sid * EPT

        def load(w, buf):
            pltpu.async_copy(dst_hbm.at[sid, w, 0], idx_v.at[buf], lsems[buf])
            pltpu.async_copy(pay_hbm.at[slab, pl.ds(ebase + w * SW, SW)],
                             upd_v.at[buf], lsems[buf])

        def drain_load(buf):
            pltpu.make_async_copy(dst_hbm.at[sid, 0, 0],
                                  idx_v.at[buf], lsems[buf]).wait()
            pltpu.make_async_copy(pay_hbm.at[slab, pl.ds(ebase, SW)],
                                  upd_v.at[buf], lsems[buf]).wait()

        load(0, 0)

        def win(w, buf, do_load):
            if do_load is True:
                load(w + 1, 1 - buf)
            else:
                @pl.when(do_load)
                def _():
                    load(w + 1, 1 - buf)

            drain_load(buf)
            pltpu.sync_copy(upd_v.at[buf], acc_sh.at[idx_v.at[buf]], add=True)

        def pair(g, carry):
            win(2 * g, 0, True)
            win(2 * g + 1, 1, g + 1 < SWIN // 2)
            return carry

        lax.fori_loop(0, SWIN // 2, pair, 0)

    for rnd in range(2):
        @pl.when(sid < NWRITERS)
        def _():
            pltpu.sync_copy(zeros_hbm, acc_sh.at[pl.ds(row0, RPT)])

        plsc.subcore_barrier()
        for c_ in range(NC):
            slab = 2 * rnd + c_

            @pl.when(cid == c_)
            def _(slab=slab):
                process(slab)

        plsc.subcore_barrier()
        for c_ in range(NC):
            slab = 2 * rnd + c_

            @pl.when((cid == c_) & (sid < NWRITERS))
            def _(slab=slab):
                pltpu.sync_copy(acc_sh.at[pl.ds(row0, RPT)],
                                out_hbm.at[slab, pl.ds(row0, RPT)])

        plsc.subcore_barrier()


def _update_body(x_ref, vec_ref, acc_ref, wv_ref, w3_ref, b3_ref, w4_ref,
                 b4_ref, xo_ref, vo_ref):
    xn = (x_ref[...] + acc_ref[0]) * INV_SQRT_2
    wv = wv_ref[...]
    vec_c = []
    vec1 = []
    vec2 = []
    for c in range(3):
        vc = vec_ref[:, c, :] + acc_ref[c + 1]
        vp = jnp.dot(vc, wv, preferred_element_type=F32)
        vec_c.append(vc)
        vec1.append(vp[:, :H])
        vec2.append(vp[:, H:])
    vec_dot = (vec1[0] * vec2[0] + vec1[1] * vec2[1] + vec1[2] * vec2[2]) * INV_SQRT_H
    vnorm = jnp.sqrt(vec2[0] ** 2 + vec2[1] ** 2 + vec2[2] ** 2 + 1e-8)
    w3 = w3_ref[...]
    t = (jnp.dot(xn, w3[:H], preferred_element_type=F32)
         + jnp.dot(vnorm, w3[H:], preferred_element_type=F32) + b3_ref[...])
    t = jax.nn.silu(t) * (1.0 / 0.6)
    xv = jnp.dot(t, w4_ref[...], preferred_element_type=F32) + b4_ref[...]
    xv1 = xv[:, :H]
    xv2 = xv[:, H:2 * H]
    xv3 = xv[:, 2 * H:]
    xo_ref[...] = xn + (xv1 + xv2 * vec_dot) * INV_SQRT_2
    for c in range(3):
        vo_ref[:, c, :] = vec_c[c] + xv3 * vec1[c]


def _node_update(x, vec, acc, Wv, W3, b3, W4, b4):
    TN = 2000
    grid = (N // TN,)
    return pl.pallas_call(
        _update_body,
        grid=grid,
        in_specs=[
            pl.BlockSpec((TN, H), lambda i: (i, 0)),
            pl.BlockSpec((TN, 3, H), lambda i: (i, 0, 0)),
            pl.BlockSpec((4, TN, H), lambda i: (0, i, 0)),
            pl.BlockSpec((H, 2 * H), lambda i: (0, 0)),
            pl.BlockSpec((2 * H, H), lambda i: (0, 0)),
            pl.BlockSpec((1, H), lambda i: (0, 0)),
            pl.BlockSpec((H, 3 * H), lambda i: (0, 0)),
            pl.BlockSpec((1, 3 * H), lambda i: (0, 0)),
        ],
        out_specs=[
            pl.BlockSpec((TN, H), lambda i: (i, 0)),
            pl.BlockSpec((TN, 3, H), lambda i: (i, 0, 0)),
        ],
        out_shape=[
            jax.ShapeDtypeStruct((N, H), F32),
            jax.ShapeDtypeStruct((N, 3, H), F32),
        ],
    )(x, vec, acc, Wv, W3, b3.reshape(1, H), W4, b4.reshape(1, 3 * H))


def kernel(x, vec, edge_index, edge_embed, edge_vec, ln_g, ln_b, W1, b1, W2,
           b2, Wr, br, Wv, W3, b3, W4, b4):
    src = edge_index[0].astype(jnp.int32).reshape(NW, GWIN, GW)
    dst = edge_index[1].astype(jnp.int32).reshape(NS, SWIN, 1, SW)
    vec2d = vec.reshape(N, 3 * H)

    table = _node_mlp(x, vec2d, ln_g, ln_b, W1, b1, W2, b2)
    xhvec_src = _sc_gather(table, src)
    pay = _edge_compute(edge_embed, xhvec_src, edge_vec, Wr, br)
    zeros = jnp.zeros((RPT, H), F32)
    acc = _sc_scatter(pay, dst, zeros)
    x_out, vec_out = _node_update(x, vec, acc, Wv, W3, b3, W4, b4)
    return (vec_out, x_out)


# speedup vs baseline: 1.3054x; 1.3054x over previous
import functools
import math

import jax
import jax.numpy as jnp
from jax import lax
from jax.experimental import pallas as pl
from jax.experimental.pallas import tpu as pltpu
from jax.experimental.pallas import tpu_sc as plsc

N = 10000
E = 320000
H = 128

INV_SQRT_3 = 1.0 / math.sqrt(3.0)
INV_SQRT_H = 1.0 / math.sqrt(float(H))
INV_SQRT_2 = 1.0 / math.sqrt(2.0)

F32 = jnp.float32

NC = 2
NS = 16
NW = NC * NS
NCHUNK = 1
EC = E // NCHUNK
GW = 80
SW = 80
EPW = EC // NW
GWIN = EPW // GW
EPT = EC // NS
SWIN = EPT // SW
RPT = 1000
NWRITERS = N // RPT

_sc_mesh = plsc.VectorSubcoreMesh(core_axis_name="c", subcore_axis_name="s")


def _node_mlp_body(x_ref, vec_ref, g_ref, bb_ref, w1_ref, b1_ref, w2_ref,
                   b2_ref, out_ref):
    x = x_ref[...]
    mu = jnp.mean(x, axis=-1, keepdims=True)
    var = jnp.mean((x - mu) ** 2, axis=-1, keepdims=True)
    xh = (x - mu) * lax.rsqrt(var + 1e-5) * g_ref[...] + bb_ref[...]
    h = jnp.dot(xh, w1_ref[...], preferred_element_type=F32) + b1_ref[...]
    h = jax.nn.silu(h) * (1.0 / 0.6)
    xh2 = jnp.dot(h, w2_ref[...], preferred_element_type=F32) + b2_ref[...]
    ua = lax.bitcast_convert_type(xh2, jnp.uint32) + 0x8000
    ub = lax.bitcast_convert_type(vec_ref[...], jnp.uint32) + 0x8000
    out_ref[...] = (ua & jnp.uint32(0xFFFF0000)) | (ub >> 16)


def _node_mlp(x, vec2d, ln_g, ln_b, W1, b1, W2, b2):
    TN = 2000
    grid = (N // TN,)
    return pl.pallas_call(
        _node_mlp_body,
        grid=grid,
        in_specs=[
            pl.BlockSpec((TN, H), lambda i: (i, 0)),
            pl.BlockSpec((TN, 3 * H), lambda i: (i, 0)),
            pl.BlockSpec((1, H), lambda i: (0, 0)),
            pl.BlockSpec((1, H), lambda i: (0, 0)),
            pl.BlockSpec((H, H), lambda i: (0, 0)),
            pl.BlockSpec((1, H), lambda i: (0, 0)),
            pl.BlockSpec((H, 3 * H), lambda i: (0, 0)),
            pl.BlockSpec((1, 3 * H), lambda i: (0, 0)),
        ],
        out_specs=pl.BlockSpec((TN, 3 * H), lambda i: (i, 0)),
        out_shape=jax.ShapeDtypeStruct((N, 3 * H), jnp.uint32),
    )(x, vec2d, ln_g.reshape(1, H), ln_b.reshape(1, H), W1, b1.reshape(1, H),
      W2, b2.reshape(1, 3 * H))


@functools.partial(
    pl.kernel,
    mesh=_sc_mesh,
    out_type=jax.ShapeDtypeStruct((EC, 3 * H), jnp.uint32),
    scratch_types=[
        pltpu.VMEM((GWIN, GW), jnp.int32),
        pltpu.VMEM((2, GW, 3 * H), jnp.uint32),
        pltpu.SemaphoreType.DMA,
        pltpu.SemaphoreType.DMA,
        pltpu.SemaphoreType.DMA,
        pltpu.SemaphoreType.DMA,
    ],
)
def _sc_gather(table, src_hbm, out, idx_v, rows_v, gsem0, gsem1, ssem0,
               ssem1):
    wid = lax.axis_index("s") * NC + lax.axis_index("c")
    pltpu.sync_copy(src_hbm.at[wid], idx_v)
    base = wid * EPW
    gsems = (gsem0, gsem1)
    ssems = (ssem0, ssem1)

    def start_gather(w, buf):
        pltpu.async_copy(table.at[idx_v.at[w]], rows_v.at[buf], gsems[buf])

    def drain_gather(buf):
        pltpu.make_async_copy(table.at[idx_v.at[0]], rows_v.at[buf],
                              gsems[buf]).wait()

    def drain_store(buf):
        pltpu.make_async_copy(rows_v.at[buf], out.at[pl.ds(base, GW)],
                              ssems[buf]).wait()

    def win(w, buf, guard):
        if guard is not False:
            def adv():
                drain_store(1 - buf)
                start_gather(w + 1, 1 - buf)

            if guard is True:
                adv()
            else:
                pl.when(guard)(adv)
        drain_gather(buf)
        pltpu.async_copy(rows_v.at[buf], out.at[pl.ds(base + w * GW, GW)],
                         ssems[buf])

    def pair(g, carry):
        win(2 * g, 0, g >= 1)
        win(2 * g + 1, 1, True)
        return carry

    start_gather(0, 0)
    start_gather(1, 1)
    lax.fori_loop(0, GWIN // 2, pair, 0)
    win(GWIN - 1, 0, False)
    drain_store(1)
    drain_store(0)


def _edge_body(ee_ref, xv_ref, ev_ref, wr_ref, br_ref, pay_ref):
    rbfh = jnp.dot(ee_ref[...], wr_ref[...], preferred_element_type=F32) + br_ref[...]
    xv = xv_ref[...]
    xhp = lax.bitcast_convert_type(xv & jnp.uint32(0xFFFF0000), F32)
    vecp = lax.bitcast_convert_type(xv << 16, F32)
    m = xhp * rbfh
    mx = m[:, :H]
    m2 = m[:, H:2 * H] * INV_SQRT_3
    m3 = m[:, 2 * H:]
    ev = ev_ref[...]
    pay_ref[0] = mx
    for c in range(3):
        pay_ref[c + 1] = (vecp[:, c * H:(c + 1) * H] * m2
                          + m3 * ev[:, c:c + 1]) * INV_SQRT_H


def _edge_compute(edge_embed, xhvec_src, edge_vec, Wr, br):
    TE = 1280
    grid = (EC // TE,)
    return pl.pallas_call(
        _edge_body,
        grid=grid,
        in_specs=[
            pl.BlockSpec((TE, H), lambda i: (i, 0)),
            pl.BlockSpec((TE, 3 * H), lambda i: (i, 0)),
            pl.BlockSpec((TE, 3), lambda i: (i, 0)),
            pl.BlockSpec((H, 3 * H), lambda i: (0, 0)),
            pl.BlockSpec((1, 3 * H), lambda i: (0, 0)),
        ],
        out_specs=pl.BlockSpec((4, TE, H), lambda i: (0, i, 0)),
        out_shape=jax.ShapeDtypeStruct((4, EC, H), F32),
    )(edge_embed, xhvec_src, edge_vec, Wr, br.reshape(1, 3 * H))


@functools.partial(
    pl.kernel,
    mesh=_sc_mesh,
    out_type=jax.ShapeDtypeStruct((4, N, H), F32),
    scratch_types=[
        pltpu.VMEM((2, SW), jnp.int32),
        pltpu.VMEM((2, SW, H), F32),
        pltpu.VMEM_SHARED((N, H), F32),
        pltpu.SemaphoreType.DMA,
        pltpu.SemaphoreType.DMA,
    ],
)
def _sc_scatter(pay_hbm, dst_hbm, zeros_hbm, out_hbm, idx_v, upd_v, acc_sh,
                lsem0, lsem1):
    cid = lax.axis_index("c")
    sid = lax.axis_index("s")
    row0 = sid * RPT
    lsems = (lsem0, lsem1)

    def process(slab):
        ebase = sid * EPT

        def load(w, buf):
            pltpu.async_copy(dst_hbm.at[sid, w, 0], idx_v.at[buf], lsems[buf])
            pltpu.async_copy(pay_hbm.at[slab, pl.ds(ebase + w * SW, SW)],
                             upd_v.at[buf], lsems[buf])

        def drain_load(buf):
            pltpu.make_async_copy(dst_hbm.at[sid, 0, 0],
                                  idx_v.at[buf], lsems[buf]).wait()
            pltpu.make_async_copy(pay_hbm.at[slab, pl.ds(ebase, SW)],
                                  upd_v.at[buf], lsems[buf]).wait()

        load(0, 0)

        def win(w, buf, do_load):
            if do_load is True:
                load(w + 1, 1 - buf)
            else:
                @pl.when(do_load)
                def _():
                    load(w + 1, 1 - buf)

            drain_load(buf)
            pltpu.sync_copy(upd_v.at[buf], acc_sh.at[idx_v.at[buf]], add=True)

        def pair(g, carry):
            win(2 * g, 0, True)
            win(2 * g + 1, 1, g + 1 < SWIN // 2)
            return carry

        lax.fori_loop(0, SWIN // 2, pair, 0)

    for rnd in range(2):
        @pl.when(sid < NWRITERS)
        def _():
            pltpu.sync_copy(zeros_hbm, acc_sh.at[pl.ds(row0, RPT)])

        plsc.subcore_barrier()
        for c_ in range(NC):
            slab = 2 * rnd + c_

            @pl.when(cid == c_)
            def _(slab=slab):
                process(slab)

        plsc.subcore_barrier()
        for c_ in range(NC):
            slab = 2 * rnd + c_

            @pl.when((cid == c_) & (sid < NWRITERS))
            def _(slab=slab):
                pltpu.sync_copy(acc_sh.at[pl.ds(row0, RPT)],
                                out_hbm.at[slab, pl.ds(row0, RPT)])

        plsc.subcore_barrier()


def _update_body(x_ref, vec_ref, acc_ref, wv_ref, w3_ref, b3_ref, w4_ref,
                 b4_ref, xo_ref, vo_ref):
    xn = (x_ref[...] + acc_ref[0]) * INV_SQRT_2
    wv = wv_ref[...]
    vec_c = []
    vec1 = []
    vec2 = []
    for c in range(3):
        vc = vec_ref[:, c, :] + acc_ref[c + 1]
        vp = jnp.dot(vc, wv, preferred_element_type=F32)
        vec_c.append(vc)
        vec1.append(vp[:, :H])
        vec2.append(vp[:, H:])
    vec_dot = (vec1[0] * vec2[0] + vec1[1] * vec2[1] + vec1[2] * vec2[2]) * INV_SQRT_H
    vnorm = jnp.sqrt(vec2[0] ** 2 + vec2[1] ** 2 + vec2[2] ** 2 + 1e-8)
    w3 = w3_ref[...]
    t = (jnp.dot(xn, w3[:H], preferred_element_type=F32)
         + jnp.dot(vnorm, w3[H:], preferred_element_type=F32) + b3_ref[...])
    t = jax.nn.silu(t) * (1.0 / 0.6)
    xv = jnp.dot(t, w4_ref[...], preferred_element_type=F32) + b4_ref[...]
    xv1 = xv[:, :H]
    xv2 = xv[:, H:2 * H]
    xv3 = xv[:, 2 * H:]
    xo_ref[...] = xn + (xv1 + xv2 * vec_dot) * INV_SQRT_2
    for c in range(3):
        vo_ref[:, c, :] = vec_c[c] + xv3 * vec1[c]


def _node_update(x, vec, acc, Wv, W3, b3, W4, b4):
    TN = 2000
    grid = (N // TN,)
    return pl.pallas_call(
        _update_body,
        grid=grid,
        in_specs=[
            pl.BlockSpec((TN, H), lambda i: (i, 0)),
            pl.BlockSpec((TN, 3, H), lambda i: (i, 0, 0)),
            pl.BlockSpec((4, TN, H), lambda i: (0, i, 0)),
            pl.BlockSpec((H, 2 * H), lambda i: (0, 0)),
            pl.BlockSpec((2 * H, H), lambda i: (0, 0)),
            pl.BlockSpec((1, H), lambda i: (0, 0)),
            pl.BlockSpec((H, 3 * H), lambda i: (0, 0)),
            pl.BlockSpec((1, 3 * H), lambda i: (0, 0)),
        ],
        out_specs=[
            pl.BlockSpec((TN, H), lambda i: (i, 0)),
            pl.BlockSpec((TN, 3, H), lambda i: (i, 0, 0)),
        ],
        out_shape=[
            jax.ShapeDtypeStruct((N, H), F32),
            jax.ShapeDtypeStruct((N, 3, H), F32),
        ],
    )(x, vec, acc, Wv, W3, b3.reshape(1, H), W4, b4.reshape(1, 3 * H))


def kernel(x, vec, edge_index, edge_embed, edge_vec, ln_g, ln_b, W1, b1, W2,
           b2, Wr, br, Wv, W3, b3, W4, b4):
    src = edge_index[0].astype(jnp.int32).reshape(NW, GWIN, GW)
    dst = edge_index[1].astype(jnp.int32).reshape(NS, SWIN, 1, SW)
    vec2d = vec.reshape(N, 3 * H)

    table = _node_mlp(x, vec2d, ln_g, ln_b, W1, b1, W2, b2)
    xhvec_src = _sc_gather(table, src)
    pay = _edge_compute(edge_embed, xhvec_src, edge_vec, Wr, br)
    zeros = jnp.zeros((RPT, H), F32)
    acc = _sc_scatter(pay, dst, zeros)
    x_out, vec_out = _node_update(x, vec, acc, Wv, W3, b3, W4, b4)
    return (vec_out, x_out)
